# Initial kernel scaffold; baseline (speedup 1.0000x reference)
import jax
import jax.numpy as jnp
from jax.experimental import pallas as pl


def kernel(input, W1, b1, W2, b2, Wr1, br1, Wr2, br2):
    raise NotImplementedError("write your pallas kernel here")


# speedup vs baseline: 4.4084x; 4.4084x over previous
import dataclasses
import functools

import jax
import jax.numpy as jnp
from jax import lax
from jax.experimental import pallas as pl
from jax.experimental.pallas import tpu as pltpu
from jax.experimental.pallas import tpu_sc as plsc

GRID = 0.01
NB = 65536
P1, P2, P3 = 73856093, 19349663, 83492791
B, N = 16, 50000
NPAD = 51200
NCORE, NSUB, LANES = 2, 16, 16
PER_SUB = NPAD // NSUB
CHUNKS = PER_SUB // LANES
JROWS = PER_SUB // 128
TW = 16
VJ = 13
VROWS = VJ * 128
ZROWS = 256
ROWS_PER_SUB = NB // NSUB
ENC_OUT = 1024
HID1 = 256
M_TILE = 2048
N_MT = NB // M_TILE


def _hash_body(pts_ref, idx_ref):
    x = pts_ref[0, 0, :]
    y = pts_ref[0, 1, :]
    z = pts_ref[0, 2, :]
    g = jnp.float32(GRID)
    cx = jnp.floor(x / g).astype(jnp.int32)
    cy = jnp.floor(y / g).astype(jnp.int32)
    cz = jnp.floor(z / g).astype(jnp.int32)
    h = (cx * P1) ^ (cy * P2) ^ (cz * P3)
    h = h & (NB - 1)
    valid = lax.broadcasted_iota(jnp.int32, (NPAD,), 0) < N
    idx_ref[0, 0, :] = jnp.where(valid, h, 0)


def _hash_call(pts_tp, interpret=False):
    return pl.pallas_call(
        _hash_body,
        grid=(B,),
        in_specs=[pl.BlockSpec((1, 3, NPAD), lambda b: (b, 0, 0))],
        out_specs=pl.BlockSpec((1, 1, NPAD), lambda b: (b, 0, 0)),
        out_shape=jax.ShapeDtypeStruct((B, 1, NPAD), jnp.int32),
        interpret=interpret,
    )(pts_tp)


def _sc_body(idx_hbm, xs_hbm, ys_hbm, zs_hbm, zeros_hbm, out_hbm,
             idxv, xv, yv, zv, vals, zbuf, table):
    c = lax.axis_index("c")
    s = lax.axis_index("s")
    iota = lax.broadcasted_iota(jnp.int32, (LANES,), 0)
    col0 = jnp.zeros((LANES,), jnp.int32)
    col1 = col0 + 1
    col2 = col0 + 2
    col3 = col0 + 3

    pltpu.sync_copy(zeros_hbm, vals)
    pltpu.sync_copy(zeros_hbm.at[pl.ds(0, ZROWS)], zbuf)

    @pl.loop(0, B // NCORE)
    def _batch(bi):
        b = c * (B // NCORE) + bi

        for k in range(ROWS_PER_SUB // ZROWS):
            pltpu.sync_copy(zbuf, table.at[pl.ds(s * ROWS_PER_SUB + k * ZROWS, ZROWS)])
        plsc.subcore_barrier()

        pltpu.sync_copy(idx_hbm.at[b, s], idxv)
        base = s * PER_SUB
        pltpu.sync_copy(xs_hbm.at[b, pl.ds(base, PER_SUB)], xv)
        pltpu.sync_copy(ys_hbm.at[b, pl.ds(base, PER_SUB)], yv)
        pltpu.sync_copy(zs_hbm.at[b, pl.ds(base, PER_SUB)], zv)

        for off, nstream in ((0, VJ), (VJ * 128, JROWS - VJ)):
            @pl.loop(0, nstream * (128 // LANES))
            def _chunk(ch):
                r0 = ch * LANES
                g0 = off + r0
                px = xv[pl.ds(g0, LANES)]
                py = yv[pl.ds(g0, LANES)]
                pz = zv[pl.ds(g0, LANES)]
                gidx = base + g0 + iota
                cnt = jnp.where(gidx < N, jnp.float32(1.0), jnp.float32(0.0))
                rows = r0 + iota
                plsc.store_scatter(vals, [rows, col0], cnt)
                plsc.store_scatter(vals, [rows, col1], px)
                plsc.store_scatter(vals, [rows, col2], py)
                plsc.store_scatter(vals, [rows, col3], pz)

            @pl.loop(0, nstream)
            def _stream(j):
                pltpu.sync_copy(vals.at[pl.ds(j * 128, 128)],
                                table.at[idxv.at[off // 128 + j]], add=True)
        plsc.subcore_barrier()

        r = s * ROWS_PER_SUB
        pltpu.sync_copy(table.at[pl.ds(r, ROWS_PER_SUB)],
                        out_hbm.at[b, pl.ds(r, ROWS_PER_SUB)])
        plsc.subcore_barrier()


@functools.lru_cache(maxsize=1)
def _sc_call():
    mesh = plsc.VectorSubcoreMesh(core_axis_name="c", subcore_axis_name="s")
    cp = pltpu.CompilerParams()
    if "needs_layout_passes" in pltpu.CompilerParams.__dataclass_fields__:
        cp = dataclasses.replace(cp, needs_layout_passes=False)
    if "use_tc_tiling_on_sc" in pltpu.CompilerParams.__dataclass_fields__:
        cp = dataclasses.replace(cp, use_tc_tiling_on_sc=False)
    return pl.kernel(
        _sc_body,
        mesh=mesh,
        compiler_params=cp,
        out_type=jax.ShapeDtypeStruct((B, NB, TW), jnp.float32),
        scratch_types=[
            pltpu.VMEM((JROWS, 128), jnp.int32),
            pltpu.VMEM((PER_SUB,), jnp.float32),
            pltpu.VMEM((PER_SUB,), jnp.float32),
            pltpu.VMEM((PER_SUB,), jnp.float32),
            pltpu.VMEM((VROWS, TW), jnp.float32),
            pltpu.VMEM((ZROWS, TW), jnp.float32),
            pltpu.VMEM_SHARED((NB, TW), jnp.float32),
        ],
    )


def _enc_body(t_ref, w1_ref, b1_ref, w2_ref, b2_ref, enc_ref, acc_ref):
    m = pl.program_id(1)

    @pl.when(m == 0)
    def _():
        acc_ref[...] = jnp.full((1, ENC_OUT), -jnp.inf, jnp.float32)

    t = t_ref[0]
    cnt = t[:, 0:1]
    colid = lax.broadcasted_iota(jnp.int32, (1, TW), 1)
    feat = jnp.where(colid == 0, t, t / jnp.maximum(cnt, 1.0))
    hdn = jnp.dot(feat, w1_ref[...], preferred_element_type=jnp.float32)
    hdn = jnp.maximum(hdn + b1_ref[...][None, :], 0.0)
    out = jnp.dot(hdn.astype(jnp.bfloat16), w2_ref[...],
                  preferred_element_type=jnp.float32)
    masked = jnp.where(cnt > 0.0, out, -jnp.inf)
    part = jnp.max(masked, axis=0)
    acc_ref[0, :] = jnp.maximum(acc_ref[0, :], part)

    @pl.when(m == N_MT - 1)
    def _():
        enc_ref[0, 0, :] = acc_ref[0, :] + b2_ref[...]


def _enc_call(tables, w1p, b1, w2bf, b2, interpret=False):
    return pl.pallas_call(
        _enc_body,
        grid=(B, N_MT),
        in_specs=[
            pl.BlockSpec((1, M_TILE, TW), lambda b, m: (b, m, 0)),
            pl.BlockSpec((TW, HID1), lambda b, m: (0, 0)),
            pl.BlockSpec((HID1,), lambda b, m: (0,)),
            pl.BlockSpec((HID1, ENC_OUT), lambda b, m: (0, 0)),
            pl.BlockSpec((ENC_OUT,), lambda b, m: (0,)),
        ],
        out_specs=pl.BlockSpec((1, 1, ENC_OUT), lambda b, m: (b, 0, 0)),
        out_shape=jax.ShapeDtypeStruct((B, 1, ENC_OUT), jnp.float32),
        scratch_shapes=[pltpu.VMEM((1, ENC_OUT), jnp.float32)],
        interpret=interpret,
    )(tables, w1p, b1, w2bf, b2).reshape(B, ENC_OUT)


def _reg_body(enc_ref, wr1_ref, br1_ref, wr2_ref, br2_ref, out_ref):
    h = jnp.dot(enc_ref[...], wr1_ref[...], preferred_element_type=jnp.float32)
    h = jnp.maximum(h + br1_ref[...][None, :], 0.0)
    out = jnp.dot(h, wr2_ref[...], preferred_element_type=jnp.float32)
    out_ref[...] = out + br2_ref[...][None, :]


def _reg_call(enc, wr1, br1, wr2, br2, interpret=False):
    return pl.pallas_call(
        _reg_body,
        out_shape=jax.ShapeDtypeStruct((B, wr2.shape[1]), jnp.float32),
        interpret=interpret,
    )(enc, wr1, br1, wr2, br2)


def kernel(input, W1, b1, W2, b2, Wr1, br1, Wr2, br2):
    pts_t = jnp.transpose(input, (0, 2, 1))
    pts_tp = jnp.pad(pts_t, ((0, 0), (0, 0), (0, NPAD - N)))
    idx = _hash_call(pts_tp)
    idx4 = idx.reshape(B, NSUB, JROWS, 128)
    zeros_hbm = jnp.zeros((VROWS, TW), jnp.float32)
    tables = _sc_call()(idx4, pts_tp[:, 0], pts_tp[:, 1], pts_tp[:, 2],
                        zeros_hbm)
    w1p = jnp.pad(W1, ((0, TW - W1.shape[0]), (0, 0)))
    enc = _enc_call(tables, w1p, b1, W2.astype(jnp.bfloat16), b2)
    pose = _reg_call(enc, Wr1, br1, Wr2, br2)
    return pose
